# baseline (speedup 1.0000x reference)
import functools

import jax
import jax.numpy as jnp
from jax import lax
from jax.experimental import pallas as pl
from jax.experimental.pallas import tpu as pltpu
from jax.experimental.pallas import tpu_sc as plsc

HIDDEN = 64
SCALE = float(HIDDEN) ** 0.5

NC = 2
NS = 16
NW = NC * NS
LANES = 16

BLK = 2048
BB = 128


def _prep_body(nfull, tail_h, lo_ref, hi_ref, out_ref):
    lo = jnp.transpose(lo_ref[...], (1, 0))
    hi = jnp.transpose(hi_ref[...], (1, 0))
    out_ref[...] = jnp.concatenate([lo, hi], axis=1) * SCALE

    @pl.when(pl.program_id(0) == nfull)
    def _():
        out_ref[pl.ds(0, tail_h)] = jnp.concatenate(
            [lo[:tail_h], lo[tail_h:2 * tail_h]], axis=1
        ) * SCALE


def _gather_body(tokt_hbm, table_hbm, out_hbm,
                 idx0, idx1, rows0, rows1, t0, t1,
                 gsem0, gsem1, osem0, osem1):
    wid = lax.axis_index("s") * NC + lax.axis_index("c")
    n_s = tokt_hbm.shape[0]
    nbt = tokt_hbm.shape[1] // BB
    per_worker = (n_s * nbt) // NW
    base = wid * per_worker

    idx = (idx0, idx1)
    rows = (rows0, rows1)
    tbuf = (t0, t1)
    gsem = (gsem0, gsem1)
    osem = (osem0, osem1)

    iota = lax.iota(jnp.int32, LANES)
    h_q = [iota + LANES * q for q in range(HIDDEN // LANES)]


    def start(u, buf):
        s = u // nbt
        bt = lax.rem(u, nbt)
        pltpu.sync_copy(tokt_hbm.at[s, pl.ds(bt * BB, BB)], idx[buf])

        pltpu.async_copy(table_hbm.at[idx[buf]], rows[buf], gsem[buf])

    def wait_gather(buf):
        pltpu.make_async_copy(table_hbm.at[pl.ds(0, BB)], rows[buf],
                              gsem[buf]).wait()

    def wait_store(buf):
        for ht in range(8):
            pltpu.make_async_copy(out_hbm.at[0, 0, 0],
                                  tbuf[buf].at[pl.ds(0, 8), pl.ds(0, BB)],
                                  osem[buf]).wait()

    def transpose_unit(buf):
        def row_body(r, carry):
            rs = jnp.full((LANES,), r, jnp.int32)
            for q in range(HIDDEN // LANES):
                v = rows[buf][r, pl.ds(q * LANES, LANES)] * SCALE
                plsc.store_scatter(tbuf[buf], [h_q[q], rs], v)
            return carry

        lax.fori_loop(0, BB, row_body, 0, unroll=2)

    def store(u, buf):
        s = u // nbt
        bt = lax.rem(u, nbt)
        for ht in range(8):
            pltpu.async_copy(tbuf[buf].at[pl.ds(8 * ht, 8), pl.ds(0, BB)],
                             out_hbm.at[s, ht, bt], osem[buf])

    start(base, 0)

    def pair_body(p, carry):
        for buf in range(2):
            t = p * 2 + buf
            u = base + t
            nbuf = 1 - buf

            @pl.when(t >= 1)
            def _():
                wait_store(nbuf)

            @pl.when(t + 1 < per_worker)
            def _():
                start(u + 1, nbuf)

            wait_gather(buf)
            transpose_unit(buf)
            store(u, buf)
        return carry

    lax.fori_loop(0, per_worker // 2, pair_body, 0)
    wait_store((per_worker - 1) % 2)


def kernel(token, table):
    nb, ns = token.shape
    vocab = table.shape[0]
    n_ht = HIDDEN // 8
    nbt = nb // BB

    table_v = table

    mesh = plsc.VectorSubcoreMesh(core_axis_name="c", subcore_axis_name="s")
    out5d = pl.kernel(
        _gather_body,
        out_type=jax.ShapeDtypeStruct((ns, n_ht, nbt, 8, BB), jnp.float32),
        mesh=mesh,
        scratch_types=[
            pltpu.VMEM((BB,), jnp.int32),
            pltpu.VMEM((BB,), jnp.int32),
            pltpu.VMEM((BB, HIDDEN), jnp.float32),
            pltpu.VMEM((BB, HIDDEN), jnp.float32),
            pltpu.VMEM((HIDDEN, BB + 1), jnp.float32),
            pltpu.VMEM((HIDDEN, BB + 1), jnp.float32),
            pltpu.SemaphoreType.DMA,
            pltpu.SemaphoreType.DMA,
            pltpu.SemaphoreType.DMA,
            pltpu.SemaphoreType.DMA,
        ],
        compiler_params=pltpu.CompilerParams(use_tc_tiling_on_sc=False, needs_layout_passes=False),
    )(token.T, table_v)
    return out5d.transpose(2, 4, 0, 1, 3).reshape(nb, ns, HIDDEN)


# speedup vs baseline: 1.4920x; 1.4920x over previous
import functools

import jax
import jax.numpy as jnp
from jax import lax
from jax.experimental import pallas as pl
from jax.experimental.pallas import tpu as pltpu
from jax.experimental.pallas import tpu_sc as plsc

HIDDEN = 64
SCALE = float(HIDDEN) ** 0.5

NC = 2
NS = 16
NW = NC * NS
LANES = 16

BLK = 2048
BB = 128
DEPTH = 4


def _prep_body(nfull, tail_h, lo_ref, hi_ref, out_ref):
    lo = jnp.transpose(lo_ref[...], (1, 0))
    hi = jnp.transpose(hi_ref[...], (1, 0))
    out_ref[...] = jnp.concatenate([lo, hi], axis=1) * SCALE

    @pl.when(pl.program_id(0) == nfull)
    def _():
        out_ref[pl.ds(0, tail_h)] = jnp.concatenate(
            [lo[:tail_h], lo[tail_h:2 * tail_h]], axis=1
        ) * SCALE


def _gather_body(tokt_hbm, table_hbm, out_hbm,
                 idx0, idx1, idx2, idx3,
                 rows0, rows1, rows2, rows3, t0, t1,
                 gsem0, gsem1, gsem2, gsem3, osem0, osem1):
    wid = lax.axis_index("s") * NC + lax.axis_index("c")
    n_s = tokt_hbm.shape[0]
    nbt = tokt_hbm.shape[1] // BB
    per_worker = (n_s * nbt) // NW
    base = wid * per_worker

    idx = (idx0, idx1, idx2, idx3)
    rows = (rows0, rows1, rows2, rows3)
    tbuf = (t0, t1)
    gsem = (gsem0, gsem1, gsem2, gsem3)
    osem = (osem0, osem1)

    iota = lax.iota(jnp.int32, LANES)
    h_q = [iota + LANES * q for q in range(HIDDEN // LANES)]

    vocab = table_hbm.shape[0]
    kk = (vocab // 2 // BLK) * BLK
    k2 = 2 * kk
    tail_h = (vocab - k2) // 2
    c_k = jnp.full((LANES,), kk, jnp.int32)
    c_k2 = jnp.full((LANES,), k2, jnp.int32)
    c_kt = jnp.full((LANES,), k2 + tail_h, jnp.int32)
    c_a = jnp.full((LANES,), k2 - 1, jnp.int32)
    c_ta = jnp.full((LANES,), k2, jnp.int32)
    c_tb = jnp.full((LANES,), k2 + 2 * tail_h - 1, jnp.int32)
    c_zero = jnp.zeros((LANES,), jnp.int32)

    def start(u, buf):
        s = u // nbt
        bt = lax.rem(u, nbt)
        pltpu.sync_copy(tokt_hbm.at[s, pl.ds(bt * BB, BB)], idx[buf])

        def fix(i, carry):
            sl = pl.ds(i * LANES, LANES)
            v = idx[buf][sl]
            adj_m = jnp.where(v >= c_k, c_a, c_zero)
            adj_t = jnp.where(v >= c_kt, c_tb, c_ta)
            idx[buf][sl] = v + v - jnp.where(v >= c_k2, adj_t, adj_m)
            return carry

        lax.fori_loop(0, BB // LANES, fix, 0, unroll=4)
        pltpu.async_copy(table_hbm.at[idx[buf]], rows[buf], gsem[buf])

    def wait_gather(buf):
        pltpu.make_async_copy(table_hbm.at[pl.ds(0, BB)], rows[buf],
                              gsem[buf]).wait()

    def wait_store(tb):
        for ht in range(8):
            pltpu.make_async_copy(out_hbm.at[0, 0, 0],
                                  tbuf[tb].at[pl.ds(0, 8), pl.ds(0, BB)],
                                  osem[tb]).wait()

    def transpose_unit(buf, tb):
        def row_body(r, carry):
            rs = jnp.full((LANES,), r, jnp.int32)
            for q in range(HIDDEN // LANES):
                v = rows[buf][r, pl.ds(q * LANES, LANES)]
                plsc.store_scatter(tbuf[tb], [h_q[q], rs], v)
            return carry

        lax.fori_loop(0, BB, row_body, 0, unroll=8)

    def store(u, tb):
        s = u // nbt
        bt = lax.rem(u, nbt)
        for ht in range(8):
            pltpu.async_copy(tbuf[tb].at[pl.ds(8 * ht, 8), pl.ds(0, BB)],
                             out_hbm.at[s, ht, bt], osem[tb])

    for w in range(DEPTH - 1):
        start(base + w, w)

    def quad_body(p, carry):
        for buf in range(DEPTH):
            t = p * DEPTH + buf
            u = base + t
            tb = buf % 2

            @pl.when(t + DEPTH - 1 < per_worker)
            def _():
                start(u + DEPTH - 1, (buf + DEPTH - 1) % DEPTH)

            @pl.when(t >= 2)
            def _():
                wait_store(tb)

            wait_gather(buf)
            transpose_unit(buf, tb)
            store(u, tb)
        return carry

    lax.fori_loop(0, per_worker // DEPTH, quad_body, 0)
    wait_store(0)
    wait_store(1)


def kernel(token, table):
    nb, ns = token.shape
    vocab = table.shape[0]
    n_ht = HIDDEN // 8
    nbt = nb // BB

    half = vocab // 2
    nfull = half // BLK
    kk = nfull * BLK
    tail_h = (vocab - 2 * kk) // 2
    table_p = pl.pallas_call(
        functools.partial(_prep_body, nfull, tail_h),
        grid=(nfull + 1,),
        in_specs=[
            pl.BlockSpec((HIDDEN, BLK),
                         lambda j: (0, jnp.where(j == nfull, 2 * nfull, j))),
            pl.BlockSpec(
                (HIDDEN, BLK),
                lambda j: (0, jnp.where(j == nfull, 2 * nfull, j + nfull))),
        ],
        out_specs=pl.BlockSpec((BLK, 2 * HIDDEN), lambda j: (j, 0)),
        out_shape=jax.ShapeDtypeStruct((half, 2 * HIDDEN), jnp.float32),
    )(table.T, table.T)
    table_v = table_p.reshape(vocab, HIDDEN)

    mesh = plsc.VectorSubcoreMesh(core_axis_name="c", subcore_axis_name="s")
    out5d = pl.kernel(
        _gather_body,
        out_type=jax.ShapeDtypeStruct((ns, n_ht, nbt, 8, BB), jnp.float32),
        mesh=mesh,
        scratch_types=[
            pltpu.VMEM((BB,), jnp.int32),
            pltpu.VMEM((BB,), jnp.int32),
            pltpu.VMEM((BB,), jnp.int32),
            pltpu.VMEM((BB,), jnp.int32),
            pltpu.VMEM((BB, HIDDEN), jnp.float32),
            pltpu.VMEM((BB, HIDDEN), jnp.float32),
            pltpu.VMEM((BB, HIDDEN), jnp.float32),
            pltpu.VMEM((BB, HIDDEN), jnp.float32),
            pltpu.VMEM((HIDDEN, BB + 1), jnp.float32),
            pltpu.VMEM((HIDDEN, BB + 1), jnp.float32),
            pltpu.SemaphoreType.DMA,
            pltpu.SemaphoreType.DMA,
            pltpu.SemaphoreType.DMA,
            pltpu.SemaphoreType.DMA,
            pltpu.SemaphoreType.DMA,
            pltpu.SemaphoreType.DMA,
        ],
        compiler_params=pltpu.CompilerParams(use_tc_tiling_on_sc=False, needs_layout_passes=False),
    )(token.T, table_v)
    return out5d.transpose(2, 4, 0, 1, 3).reshape(nb, ns, HIDDEN)


# speedup vs baseline: 1.6206x; 1.0862x over previous
import functools

import jax
import jax.numpy as jnp
from jax import lax
from jax.experimental import pallas as pl
from jax.experimental.pallas import tpu as pltpu
from jax.experimental.pallas import tpu_sc as plsc

HIDDEN = 64
SCALE = float(HIDDEN) ** 0.5

NC = 2
NS = 16
NW = NC * NS
LANES = 16

BLK = 4096
BB = 128
DEPTH = 4


def _prep_body(nfull, tail_h, lo_ref, hi_ref, out_ref):
    lo = jnp.transpose(lo_ref[...], (1, 0))
    hi = jnp.transpose(hi_ref[...], (1, 0))
    out_ref[...] = jnp.concatenate([lo, hi], axis=1) * SCALE

    @pl.when(pl.program_id(0) == nfull)
    def _():
        out_ref[pl.ds(0, tail_h)] = jnp.concatenate(
            [lo[:tail_h], lo[tail_h:2 * tail_h]], axis=1
        ) * SCALE


def _gather_body(tokt_hbm, table_hbm, out_hbm,
                 idx0, idx1, idx2, idx3,
                 rows0, rows1, rows2, rows3, t0, t1,
                 gsem0, gsem1, gsem2, gsem3, osem0, osem1):
    wid = lax.axis_index("s") * NC + lax.axis_index("c")
    n_s = tokt_hbm.shape[0]
    nbt = tokt_hbm.shape[1] // BB
    per_worker = (n_s * nbt) // NW
    base = wid * per_worker

    idx = (idx0, idx1, idx2, idx3)
    rows = (rows0, rows1, rows2, rows3)
    tbuf = (t0, t1)
    gsem = (gsem0, gsem1, gsem2, gsem3)
    osem = (osem0, osem1)

    iota = lax.iota(jnp.int32, LANES)
    h_q = [iota + LANES * q for q in range(HIDDEN // LANES)]

    vocab = table_hbm.shape[0]
    kk = (vocab // 2 // BLK) * BLK
    k2 = 2 * kk
    tail_h = (vocab - k2) // 2
    c_k = jnp.full((LANES,), kk, jnp.int32)
    c_k2 = jnp.full((LANES,), k2, jnp.int32)
    c_kt = jnp.full((LANES,), k2 + tail_h, jnp.int32)
    c_a = jnp.full((LANES,), k2 - 1, jnp.int32)
    c_ta = jnp.full((LANES,), k2, jnp.int32)
    c_tb = jnp.full((LANES,), k2 + 2 * tail_h - 1, jnp.int32)
    c_zero = jnp.zeros((LANES,), jnp.int32)

    def start(u, buf):
        s = u // nbt
        bt = lax.rem(u, nbt)
        pltpu.sync_copy(tokt_hbm.at[s, pl.ds(bt * BB, BB)], idx[buf])

        def fix(i, carry):
            sl = pl.ds(i * LANES, LANES)
            v = idx[buf][sl]
            adj_m = jnp.where(v >= c_k, c_a, c_zero)
            adj_t = jnp.where(v >= c_kt, c_tb, c_ta)
            idx[buf][sl] = v + v - jnp.where(v >= c_k2, adj_t, adj_m)
            return carry

        lax.fori_loop(0, BB // LANES, fix, 0, unroll=4)
        pltpu.async_copy(table_hbm.at[idx[buf]], rows[buf], gsem[buf])

    def wait_gather(buf):
        pltpu.make_async_copy(table_hbm.at[pl.ds(0, BB)], rows[buf],
                              gsem[buf]).wait()

    def wait_store(tb):
        for ht in range(8):
            pltpu.make_async_copy(out_hbm.at[0, 0, 0],
                                  tbuf[tb].at[pl.ds(0, 8), pl.ds(0, BB)],
                                  osem[tb]).wait()

    def transpose_unit(buf, tb):
        def row_body(r, carry):
            rs = jnp.full((LANES,), r, jnp.int32)
            for q in range(HIDDEN // LANES):
                v = rows[buf][r, pl.ds(q * LANES, LANES)]
                plsc.store_scatter(tbuf[tb], [h_q[q], rs], v)
            return carry

        lax.fori_loop(0, BB, row_body, 0, unroll=8)

    def store(u, tb):
        s = u // nbt
        bt = lax.rem(u, nbt)
        for ht in range(8):
            pltpu.async_copy(tbuf[tb].at[pl.ds(8 * ht, 8), pl.ds(0, BB)],
                             out_hbm.at[s, ht, bt], osem[tb])

    for w in range(DEPTH - 1):
        start(base + w, w)

    def quad_body(p, carry):
        for buf in range(DEPTH):
            t = p * DEPTH + buf
            u = base + t
            tb = buf % 2

            @pl.when(t + DEPTH - 1 < per_worker)
            def _():
                start(u + DEPTH - 1, (buf + DEPTH - 1) % DEPTH)

            @pl.when(t >= 2)
            def _():
                wait_store(tb)

            wait_gather(buf)
            transpose_unit(buf, tb)
            store(u, tb)
        return carry

    lax.fori_loop(0, per_worker // DEPTH, quad_body, 0)
    wait_store(0)
    wait_store(1)


def kernel(token, table):
    nb, ns = token.shape
    vocab = table.shape[0]
    n_ht = HIDDEN // 8
    nbt = nb // BB

    half = vocab // 2
    nfull = half // BLK
    kk = nfull * BLK
    tail_h = (vocab - 2 * kk) // 2
    table_p = pl.pallas_call(
        functools.partial(_prep_body, nfull, tail_h),
        grid=(nfull + 1,),
        in_specs=[
            pl.BlockSpec((HIDDEN, BLK),
                         lambda j: (0, jnp.where(j == nfull, 2 * nfull, j))),
            pl.BlockSpec(
                (HIDDEN, BLK),
                lambda j: (0, jnp.where(j == nfull, 2 * nfull, j + nfull))),
        ],
        out_specs=pl.BlockSpec((BLK, 2 * HIDDEN), lambda j: (j, 0)),
        out_shape=jax.ShapeDtypeStruct((half, 2 * HIDDEN), jnp.float32),
    )(table.T, table.T)
    table_v = table_p.reshape(vocab, HIDDEN)

    mesh = plsc.VectorSubcoreMesh(core_axis_name="c", subcore_axis_name="s")
    out5d = pl.kernel(
        _gather_body,
        out_type=jax.ShapeDtypeStruct((ns, n_ht, nbt, 8, BB), jnp.float32),
        mesh=mesh,
        scratch_types=[
            pltpu.VMEM((BB,), jnp.int32),
            pltpu.VMEM((BB,), jnp.int32),
            pltpu.VMEM((BB,), jnp.int32),
            pltpu.VMEM((BB,), jnp.int32),
            pltpu.VMEM((BB, HIDDEN), jnp.float32),
            pltpu.VMEM((BB, HIDDEN), jnp.float32),
            pltpu.VMEM((BB, HIDDEN), jnp.float32),
            pltpu.VMEM((BB, HIDDEN), jnp.float32),
            pltpu.VMEM((HIDDEN, BB + 1), jnp.float32),
            pltpu.VMEM((HIDDEN, BB + 1), jnp.float32),
            pltpu.SemaphoreType.DMA,
            pltpu.SemaphoreType.DMA,
            pltpu.SemaphoreType.DMA,
            pltpu.SemaphoreType.DMA,
            pltpu.SemaphoreType.DMA,
            pltpu.SemaphoreType.DMA,
        ],
        compiler_params=pltpu.CompilerParams(use_tc_tiling_on_sc=False, needs_layout_passes=False),
    )(token.T, table_v)
    return out5d.transpose(2, 4, 0, 1, 3).reshape(nb, ns, HIDDEN)
